# Initial kernel scaffold; baseline (speedup 1.0000x reference)
import jax
import jax.numpy as jnp
from jax.experimental import pallas as pl


def kernel(patches):
    raise NotImplementedError("write your pallas kernel here")


# speedup vs baseline: 19.8724x; 19.8724x over previous
import functools

import jax
import jax.numpy as jnp
import numpy as np
from jax import lax
from jax.experimental import pallas as pl
from jax.experimental.pallas import tpu as pltpu
from jax.experimental.pallas import tpu_sc as plsc

_B, _T, _C = 64, 576, 768
_ROWS = _B * _T

_FWD = np.array([
    121,480,35,130,263,557,148,197,410,398,45,520,176,569,462,446,366,575,257,179,139,315,501,188,
    312,499,318,448,304,99,309,567,144,152,517,189,487,552,544,516,325,31,112,532,518,495,356,493,
    507,543,268,429,538,409,541,85,63,117,417,174,565,441,509,525,481,272,114,254,564,524,82,65,7,
    350,4,101,463,452,444,102,78,163,157,302,183,29,240,177,278,259,108,553,305,83,129,367,212,277,
    504,300,44,211,16,58,123,562,37,336,111,19,61,540,447,2,142,34,542,369,339,551,156,436,5,461,
    415,90,363,514,175,167,284,379,251,110,72,155,178,323,291,388,269,535,354,573,533,368,219,510,
    153,30,275,42,186,342,406,468,439,307,256,419,246,3,362,380,327,393,70,566,378,400,271,522,488,
    311,67,273,223,422,39,56,274,192,169,349,218,195,476,173,245,241,69,383,80,22,571,6,321,199,
    345,118,235,54,442,479,423,266,77,425,147,18,340,298,249,294,375,382,10,570,11,234,53,236,455,
    528,94,515,332,511,331,437,353,489,287,32,217,283,355,529,407,159,440,15,470,184,49,548,137,50,
    558,138,20,563,549,445,237,280,253,185,527,460,43,389,335,561,258,370,344,92,8,503,324,140,233,
    24,81,239,314,453,96,475,467,154,135,472,490,469,559,500,264,160,106,128,265,426,386,191,9,200,
    40,187,71,346,438,333,248,164,207,93,59,201,158,210,420,402,75,508,131,411,97,66,25,196,424,
    364,497,242,338,206,243,397,341,450,414,238,560,295,432,431,308,73,512,320,13,52,556,491,203,
    289,303,202,255,194,88,250,337,62,230,150,261,330,262,209,132,357,87,76,198,486,60,244,457,47,
    392,374,276,33,79,451,180,403,247,14,459,286,421,458,228,17,38,86,550,231,190,232,545,482,23,
    536,105,484,395,427,301,474,376,555,405,546,494,471,391,574,534,313,220,0,473,145,371,213,226,
    381,133,281,41,64,572,416,21,443,161,279,285,166,124,116,449,26,165,168,193,57,208,181,89,146,
    182,126,125,297,1,115,28,113,530,225,361,351,537,465,172,377,162,48,170,466,505,227,36,252,502,
    492,521,119,151,385,306,120,372,390,224,523,122,270,100,568,418,433,329,365,396,526,91,519,222,
    55,496,498,103,51,293,215,384,127,98,483,506,282,107,27,322,74,136,229,319,328,531,430,343,204,
    221,296,12,134,454,477,554,408,109,84,539,428,317,513,358,394,299,205,171,288,143,68,267,216,
    435,547,149,485,434,141,464,334,404,104,352,95,387,316,214,290,46,310,348,401,260,478,292,359,
    326,347,456,399,373,412,360,413,
], dtype=np.int32)
_BWD = np.argsort(_FWD).astype(np.int32)

_SRC_ROWS = (
    np.arange(_B, dtype=np.int32)[:, None] * _T + _FWD[None, :]
).reshape(-1)

_NC, _NS = 2, 16
_NW = _NC * _NS
_RPW = _ROWS // _NW
_CHUNK = 128
_NCHUNK = _RPW // _CHUNK


@functools.partial(
    pl.kernel,
    mesh=plsc.VectorSubcoreMesh(core_axis_name="c", subcore_axis_name="s"),
    out_type=jax.ShapeDtypeStruct((_ROWS, _C), jnp.float32),
    scratch_types=[
        pltpu.VMEM((_RPW,), jnp.int32),
        pltpu.VMEM((_CHUNK, _C), jnp.float32),
        pltpu.SemaphoreType.DMA,
    ],
)
def _shuffle_rows(p_hbm, idx_hbm, out_hbm, idx_v, rows_v, sem):
    wid = lax.axis_index("s") * _NC + lax.axis_index("c")
    base = wid * _RPW
    pltpu.sync_copy(idx_hbm.at[pl.ds(base, _RPW)], idx_v)
    for ci in range(_NCHUNK):
        pltpu.async_copy(
            p_hbm.at[idx_v.at[pl.ds(ci * _CHUNK, _CHUNK)]], rows_v, sem
        ).wait()
        pltpu.sync_copy(rows_v, out_hbm.at[pl.ds(base + ci * _CHUNK, _CHUNK)])


def kernel(patches):
    flat = patches.reshape(_ROWS, _C)
    shuffled = _shuffle_rows(flat, jnp.asarray(_SRC_ROWS)).reshape(_B, _T, _C)
    fwd_b = jnp.broadcast_to(jnp.asarray(_FWD)[None, :], (_B, _T))
    bwd_b = jnp.broadcast_to(jnp.asarray(_BWD)[None, :], (_B, _T))
    return (shuffled, fwd_b, bwd_b)


# speedup vs baseline: 20.8168x; 1.0475x over previous
import functools

import jax
import jax.numpy as jnp
import numpy as np
from jax import lax
from jax.experimental import pallas as pl
from jax.experimental.pallas import tpu as pltpu
from jax.experimental.pallas import tpu_sc as plsc

_B, _T, _C = 64, 576, 768
_ROWS = _B * _T

_FWD = np.array([
    121,480,35,130,263,557,148,197,410,398,45,520,176,569,462,446,366,575,257,179,139,315,501,188,
    312,499,318,448,304,99,309,567,144,152,517,189,487,552,544,516,325,31,112,532,518,495,356,493,
    507,543,268,429,538,409,541,85,63,117,417,174,565,441,509,525,481,272,114,254,564,524,82,65,7,
    350,4,101,463,452,444,102,78,163,157,302,183,29,240,177,278,259,108,553,305,83,129,367,212,277,
    504,300,44,211,16,58,123,562,37,336,111,19,61,540,447,2,142,34,542,369,339,551,156,436,5,461,
    415,90,363,514,175,167,284,379,251,110,72,155,178,323,291,388,269,535,354,573,533,368,219,510,
    153,30,275,42,186,342,406,468,439,307,256,419,246,3,362,380,327,393,70,566,378,400,271,522,488,
    311,67,273,223,422,39,56,274,192,169,349,218,195,476,173,245,241,69,383,80,22,571,6,321,199,
    345,118,235,54,442,479,423,266,77,425,147,18,340,298,249,294,375,382,10,570,11,234,53,236,455,
    528,94,515,332,511,331,437,353,489,287,32,217,283,355,529,407,159,440,15,470,184,49,548,137,50,
    558,138,20,563,549,445,237,280,253,185,527,460,43,389,335,561,258,370,344,92,8,503,324,140,233,
    24,81,239,314,453,96,475,467,154,135,472,490,469,559,500,264,160,106,128,265,426,386,191,9,200,
    40,187,71,346,438,333,248,164,207,93,59,201,158,210,420,402,75,508,131,411,97,66,25,196,424,
    364,497,242,338,206,243,397,341,450,414,238,560,295,432,431,308,73,512,320,13,52,556,491,203,
    289,303,202,255,194,88,250,337,62,230,150,261,330,262,209,132,357,87,76,198,486,60,244,457,47,
    392,374,276,33,79,451,180,403,247,14,459,286,421,458,228,17,38,86,550,231,190,232,545,482,23,
    536,105,484,395,427,301,474,376,555,405,546,494,471,391,574,534,313,220,0,473,145,371,213,226,
    381,133,281,41,64,572,416,21,443,161,279,285,166,124,116,449,26,165,168,193,57,208,181,89,146,
    182,126,125,297,1,115,28,113,530,225,361,351,537,465,172,377,162,48,170,466,505,227,36,252,502,
    492,521,119,151,385,306,120,372,390,224,523,122,270,100,568,418,433,329,365,396,526,91,519,222,
    55,496,498,103,51,293,215,384,127,98,483,506,282,107,27,322,74,136,229,319,328,531,430,343,204,
    221,296,12,134,454,477,554,408,109,84,539,428,317,513,358,394,299,205,171,288,143,68,267,216,
    435,547,149,485,434,141,464,334,404,104,352,95,387,316,214,290,46,310,348,401,260,478,292,359,
    326,347,456,399,373,412,360,413,
], dtype=np.int32)
_BWD = np.argsort(_FWD).astype(np.int32)

_SRC_ROWS = (
    np.arange(_B, dtype=np.int32)[:, None] * _T + _FWD[None, :]
).reshape(-1)

_NC, _NS = 2, 16
_NW = _NC * _NS
_RPW = _ROWS // _NW
_CHUNK = 72
_NCHUNK = _RPW // _CHUNK


@functools.partial(
    pl.kernel,
    mesh=plsc.VectorSubcoreMesh(core_axis_name="c", subcore_axis_name="s"),
    out_type=jax.ShapeDtypeStruct((_ROWS, _C), jnp.float32),
    scratch_types=[
        pltpu.VMEM((_RPW,), jnp.int32),
        pltpu.VMEM((_CHUNK, _C), jnp.float32),
        pltpu.VMEM((_CHUNK, _C), jnp.float32),
        pltpu.SemaphoreType.DMA,
        pltpu.SemaphoreType.DMA,
        pltpu.SemaphoreType.DMA,
        pltpu.SemaphoreType.DMA,
    ],
)
def _shuffle_rows(p_hbm, idx_hbm, out_hbm, idx_v, buf0, buf1, gs0, gs1, os0, os1):
    bufs, gsem, osem = (buf0, buf1), (gs0, gs1), (os0, os1)
    wid = lax.axis_index("s") * _NC + lax.axis_index("c")
    base = wid * _RPW
    pltpu.sync_copy(idx_hbm.at[pl.ds(base, _RPW)], idx_v)

    def start_gather(ci):
        b = ci % 2
        return pltpu.async_copy(
            p_hbm.at[idx_v.at[pl.ds(ci * _CHUNK, _CHUNK)]], bufs[b], gsem[b]
        )

    gd = [None, None]
    od = [None, None]
    gd[0] = start_gather(0)
    for ci in range(_NCHUNK):
        b = ci % 2
        if ci + 1 < _NCHUNK:
            nb = (ci + 1) % 2
            if od[nb] is not None:
                od[nb].wait()
            gd[nb] = start_gather(ci + 1)
        gd[b].wait()
        od[b] = pltpu.async_copy(
            bufs[b], out_hbm.at[pl.ds(base + ci * _CHUNK, _CHUNK)], osem[b]
        )
    od[(_NCHUNK - 2) % 2].wait()
    od[(_NCHUNK - 1) % 2].wait()


def kernel(patches):
    flat = patches.reshape(_ROWS, _C)
    shuffled = _shuffle_rows(flat, jnp.asarray(_SRC_ROWS)).reshape(_B, _T, _C)
    fwd_b = jnp.broadcast_to(jnp.asarray(_FWD)[None, :], (_B, _T))
    bwd_b = jnp.broadcast_to(jnp.asarray(_BWD)[None, :], (_B, _T))
    return (shuffled, fwd_b, bwd_b)
